# baseline (speedup 1.0000x reference)
import functools

import jax
import jax.numpy as jnp
from jax import lax
from jax.experimental import pallas as pl
from jax.experimental.pallas import tpu as pltpu
from jax.experimental.pallas import tpu_sc as plsc

_E = 320000
_D = 128
_NB, _NS, _NC = 22, 6, 2
_NCOMBO = _NB * _NS * _NC

_NCORES = 2
_NSUB = 16
_NW = _NCORES * _NSUB
_EPW = _E // _NW
_L = 16
_CH = 80
_NCHUNK = _EPW // _CH
_NBUF = 8
_LAG = 2


def _table_body(b_ref, s_ref, c_ref, t_ref):
    rid_b = lax.broadcasted_iota(jnp.int32, (_NCOMBO, _NB), 0)
    cid_b = lax.broadcasted_iota(jnp.int32, (_NCOMBO, _NB), 1)
    ohb = (rid_b // (_NS * _NC) == cid_b).astype(jnp.float32)
    rid_s = lax.broadcasted_iota(jnp.int32, (_NCOMBO, _NS), 0)
    cid_s = lax.broadcasted_iota(jnp.int32, (_NCOMBO, _NS), 1)
    ohs = ((rid_s // _NC) % _NS == cid_s).astype(jnp.float32)
    rid_c = lax.broadcasted_iota(jnp.int32, (_NCOMBO, _NC), 0)
    cid_c = lax.broadcasted_iota(jnp.int32, (_NCOMBO, _NC), 1)
    ohc = (rid_c % _NC == cid_c).astype(jnp.float32)
    t_ref[...] = (
        jnp.dot(ohb, b_ref[...], preferred_element_type=jnp.float32)
        + jnp.dot(ohs, s_ref[...], preferred_element_type=jnp.float32)
        + jnp.dot(ohc, c_ref[...], preferred_element_type=jnp.float32)
    )


def _codes_body(a_ref, b_ref, c_ref, o_ref):
    o_ref[...] = (a_ref[...] * (_NS * _NC) + b_ref[...] * _NC
                  + c_ref[...]) * _D


def _build_codes(ea0, ea1, ea2):
    r = _E // _D
    out = pl.pallas_call(
        _codes_body,
        out_shape=jax.ShapeDtypeStruct((r, _D), jnp.int32),
    )(ea0.reshape(r, _D), ea1.reshape(r, _D), ea2.reshape(r, _D))
    return out.reshape(_E)


def _build_table(bond, stereo, conj):
    return pl.pallas_call(
        _table_body,
        out_shape=jax.ShapeDtypeStruct((_NCOMBO, _D), jnp.float32),
    )(bond, stereo, conj)


def _sc_body(t_hbm, code_hbm, out_hbm,
             t_v, code_v, rows_v, wr_sem):
    wid = lax.axis_index("s") * _NCORES + lax.axis_index("c")
    base = wid * _EPW

    pltpu.sync_copy(t_hbm, t_v)
    pltpu.sync_copy(code_hbm.at[pl.ds(base, _EPW)], code_v)

    def w_desc(g, b):
        return pltpu.make_async_copy(
            rows_v.at[pl.ds(b * _CH, _CH)],
            out_hbm.at[pl.ds(base + g * _CH, _CH)],
            wr_sem.at[b])

    def chunk_step(g, carry):
        b = lax.rem(g, _NBUF)

        def reclaim(j):
            w_desc(j, lax.rem(j, _NBUF)).wait()
            return 0

        lax.cond(g >= _NBUF, reclaim, lambda j: 0, g - _NBUF)

        @plsc.parallel_loop(0, _CH // _L, unroll=2)
        def fill_group(e16):
            codes = code_v[pl.ds(g * _CH + e16 * _L, _L)]
            for i in range(_L):
                sc = codes[i]
                row = b * _CH + e16 * _L + i
                for d in range(8):
                    rows_v[row, pl.ds(d * _L, _L)] = t_v[pl.ds(sc + d * _L, _L)]

        w_desc(g, b).start()
        return carry

    lax.fori_loop(0, _NCHUNK, chunk_step, 0)

    for j in range(_NCHUNK - _NBUF, _NCHUNK):
        w_desc(j, j % _NBUF).wait()


_sc_gather = functools.partial(
    pl.kernel,
    out_type=jax.ShapeDtypeStruct((_E, _D), jnp.float32),
    mesh=plsc.VectorSubcoreMesh(core_axis_name="c", subcore_axis_name="s"),
    compiler_params=pltpu.CompilerParams(needs_layout_passes=False),
    scratch_types=[
        pltpu.VMEM((_NCOMBO * _D,), jnp.float32),
        pltpu.VMEM((_EPW,), jnp.int32),
        pltpu.VMEM((_NBUF * _CH, _D), jnp.float32),
        pltpu.SemaphoreType.DMA((_NBUF,)),
    ],
)(_sc_body)


@jax.jit
def kernel(edge_attr, bond_embedding, stereo_embedding, conj_embedding):
    t = _build_table(bond_embedding, stereo_embedding, conj_embedding)
    ea0 = edge_attr[:, 0].astype(jnp.int32)
    ea1 = edge_attr[:, 1].astype(jnp.int32)
    ea2 = edge_attr[:, 2].astype(jnp.int32)
    codes = _build_codes(ea0, ea1, ea2)
    return _sc_gather(t.reshape(-1), codes)
